# baseline (speedup 1.0000x reference)
import jax
import jax.numpy as jnp
from jax.experimental import pallas as pl

EPS = 1e-5


def _gcn_conv(x, edge_index, W, b):
    n = x.shape[0]
    loop = jnp.arange(n, dtype=edge_index.dtype)
    src = jnp.concatenate([edge_index[0], loop])
    dst = jnp.concatenate([edge_index[1], loop])
    xw = x @ W
    ones = jnp.ones(dst.shape[0], dtype=x.dtype)
    deg = jax.ops.segment_sum(ones, dst, num_segments=n)
    dinv = jnp.where(deg > 0, deg ** -0.5, 0.0)
    norm = dinv[src] * dinv[dst]
    msgs = xw[src] * norm[:, None]
    out = jax.ops.segment_sum(msgs, dst, num_segments=n)
    return out + b


def _bn(x, gamma, beta):
    mean = jnp.mean(x, axis=0)
    var = jnp.var(x, axis=0)
    return (x - mean) / jnp.sqrt(var + EPS) * gamma + beta


def _copy_kernel(x_ref, o_ref):
    o_ref[...] = x_ref[...]


def kernel(x, edge_index, W1, b1, g1, be1, W2, b2, g2, be2, W3, b3):
    h = _gcn_conv(x, edge_index, W1, b1)
    h = jax.nn.relu(_bn(h, g1, be1))
    h = _gcn_conv(h, edge_index, W2, b2)
    h = jax.nn.relu(_bn(h, g2, be2))
    out = _gcn_conv(h, edge_index, W3, b3)
    out = pl.pallas_call(
        _copy_kernel,
        out_shape=jax.ShapeDtypeStruct(out.shape, out.dtype),
    )(out)
    return out


# speedup vs baseline: 6.5494x; 6.5494x over previous
import functools

import jax
import jax.numpy as jnp
from jax import lax
from jax.experimental import pallas as pl
from jax.experimental.pallas import tpu as pltpu
from jax.experimental.pallas import tpu_sc as plsc

N = 10000
D = 128
E = 320000
EPS = 1e-5

NC = 2
NS = 16
NW = NC * NS

BLK = 128
BPW = 80
CHUNK = 16
NCHUNK = BPW // CHUNK
EPW = BPW * BLK
E_PAD = NW * EPW

NPAD = 10240
RPT = NPAD // NS
RB = 512
GB = NPAD // RB

_mesh = plsc.VectorSubcoreMesh(core_axis_name="c", subcore_axis_name="s")


@functools.partial(
    pl.kernel,
    out_type=jax.ShapeDtypeStruct((NC, NPAD), jnp.float32),
    mesh=_mesh,
    scratch_types=[
        pltpu.VMEM_SHARED((NPAD,), jnp.float32),
        pltpu.VMEM((BPW, BLK), jnp.int32),
        pltpu.VMEM((BLK,), jnp.float32),
    ],
)
def _sc_degree(ones_hbm, zeros_hbm, dstp_hbm, out_hbm, dacc, didx, ones_v):
    c = lax.axis_index("c")
    s = lax.axis_index("s")
    w = c * NS + s
    pltpu.sync_copy(dstp_hbm.at[w], didx)
    pltpu.sync_copy(ones_hbm, ones_v)
    pltpu.sync_copy(zeros_hbm, dacc.at[pl.ds(s * RPT, RPT)])
    plsc.subcore_barrier()

    def body(b, carry):
        pltpu.sync_copy(ones_v, dacc.at[didx.at[b]], add=True)
        return carry

    lax.fori_loop(0, BPW, body, 0)
    plsc.subcore_barrier()
    pltpu.sync_copy(dacc.at[pl.ds(s * RPT, RPT)], out_hbm.at[c, pl.ds(s * RPT, RPT)])


@functools.partial(
    pl.kernel,
    out_type=jax.ShapeDtypeStruct((NC, NPAD, D), jnp.float32),
    mesh=_mesh,
    scratch_types=[
        pltpu.VMEM_SHARED((NPAD, D), jnp.float32),
        pltpu.VMEM((CHUNK, BLK), jnp.int32),
        pltpu.VMEM((CHUNK, BLK), jnp.int32),
        pltpu.VMEM((BLK, D), jnp.float32),
        pltpu.VMEM((BLK, D), jnp.float32),
        pltpu.SemaphoreType.DMA,
        pltpu.SemaphoreType.DMA,
    ],
)
def _sc_aggregate(table_hbm, zeros_hbm, srcp_hbm, dstp_hbm, out_hbm,
                  accum, sidx, didx, buf0, buf1, sem0, sem1):
    c = lax.axis_index("c")
    s = lax.axis_index("s")
    w = c * NS + s
    pltpu.sync_copy(zeros_hbm, accum.at[pl.ds(s * RPT, RPT), :])
    plsc.subcore_barrier()

    def chunk_body(k, carry):
        pltpu.sync_copy(srcp_hbm.at[w, pl.ds(k * CHUNK, CHUNK)], sidx)
        pltpu.sync_copy(dstp_hbm.at[w, pl.ds(k * CHUNK, CHUNK)], didx)
        pltpu.async_copy(table_hbm.at[sidx.at[0]], buf0, sem0)

        def body(i, carry2):
            b1 = 2 * i + 1
            b2 = jnp.minimum(2 * i + 2, CHUNK - 1)
            pltpu.async_copy(table_hbm.at[sidx.at[b1]], buf1, sem1)
            pltpu.make_async_copy(table_hbm.at[sidx.at[0]], buf0, sem0).wait()
            pltpu.sync_copy(buf0, accum.at[didx.at[2 * i]], add=True)
            pltpu.async_copy(table_hbm.at[sidx.at[b2]], buf0, sem0)
            pltpu.make_async_copy(table_hbm.at[sidx.at[0]], buf1, sem1).wait()
            pltpu.sync_copy(buf1, accum.at[didx.at[b1]], add=True)
            return carry2

        lax.fori_loop(0, CHUNK // 2, body, 0)
        pltpu.make_async_copy(table_hbm.at[sidx.at[0]], buf0, sem0).wait()
        return carry

    lax.fori_loop(0, NCHUNK, chunk_body, 0)
    plsc.subcore_barrier()
    pltpu.sync_copy(accum.at[pl.ds(s * RPT, RPT), :],
                    out_hbm.at[c, pl.ds(s * RPT, RPT), :])


def _t1_body(x_ref, w_ref, deg_ref, o_ref):
    dinv = lax.rsqrt(deg_ref[0] + deg_ref[1] + 1.0)
    o_ref[...] = jnp.dot(x_ref[...], w_ref[...],
                         preferred_element_type=jnp.float32) * dinv


def _t1(x_pad, W, deg):
    return pl.pallas_call(
        _t1_body,
        grid=(GB,),
        in_specs=[
            pl.BlockSpec((RB, D), lambda i: (i, 0)),
            pl.BlockSpec((D, D), lambda i: (0, 0)),
            pl.BlockSpec((NC, RB, 1), lambda i: (0, i, 0)),
        ],
        out_specs=pl.BlockSpec((RB, D), lambda i: (i, 0)),
        out_shape=jax.ShapeDtypeStruct((NPAD, D), jnp.float32),
    )(x_pad, W, deg)


def _tmid_body(p_ref, xwp_ref, deg_ref, b_ref, g_ref, be_ref, wn_ref,
               o_ref, stats):
    t = pl.program_id(0)
    i = pl.program_id(1)
    dinv = lax.rsqrt(deg_ref[0] + deg_ref[1] + 1.0)
    y = (p_ref[0] + p_ref[1] + xwp_ref[...]) * dinv + b_ref[...]
    rows = jax.lax.broadcasted_iota(jnp.int32, (RB, 1), 0) + i * RB
    mask = rows < N

    @pl.when(t == 0)
    def _():
        @pl.when(i == 0)
        def _():
            stats[...] = jnp.zeros_like(stats)
        ym = jnp.where(mask, y, 0.0)
        stats[0:1, :] += jnp.sum(ym, axis=0, keepdims=True)
        stats[1:2, :] += jnp.sum(ym * ym, axis=0, keepdims=True)

    @pl.when(t == 1)
    def _():
        mean = stats[0:1, :] * (1.0 / N)
        var = stats[1:2, :] * (1.0 / N) - mean * mean
        rstd = lax.rsqrt(var + EPS)
        h = jnp.maximum((y - mean) * rstd * g_ref[...] + be_ref[...], 0.0)
        nxt = jnp.dot(h, wn_ref[...], preferred_element_type=jnp.float32)
        o_ref[...] = jnp.where(mask, nxt * dinv, 0.0)


def _tmid(p, xwp, deg, b, g, be, Wn):
    return pl.pallas_call(
        _tmid_body,
        grid=(2, GB),
        in_specs=[
            pl.BlockSpec((NC, RB, D), lambda t, i: (0, i, 0)),
            pl.BlockSpec((RB, D), lambda t, i: (i, 0)),
            pl.BlockSpec((NC, RB, 1), lambda t, i: (0, i, 0)),
            pl.BlockSpec((1, D), lambda t, i: (0, 0)),
            pl.BlockSpec((1, D), lambda t, i: (0, 0)),
            pl.BlockSpec((1, D), lambda t, i: (0, 0)),
            pl.BlockSpec((D, D), lambda t, i: (0, 0)),
        ],
        out_specs=pl.BlockSpec((RB, D), lambda t, i: (i, 0)),
        out_shape=jax.ShapeDtypeStruct((NPAD, D), jnp.float32),
        scratch_shapes=[pltpu.VMEM((8, D), jnp.float32)],
    )(p, xwp, deg, b, g, be, Wn)


def _tlast_body(p_ref, xwp_ref, deg_ref, b_ref, o_ref):
    dinv = lax.rsqrt(deg_ref[0] + deg_ref[1] + 1.0)
    o_ref[...] = (p_ref[0] + p_ref[1] + xwp_ref[...]) * dinv + b_ref[...]


def _tlast(p, xwp, deg, b):
    return pl.pallas_call(
        _tlast_body,
        grid=(GB,),
        in_specs=[
            pl.BlockSpec((NC, RB, D), lambda i: (0, i, 0)),
            pl.BlockSpec((RB, D), lambda i: (i, 0)),
            pl.BlockSpec((NC, RB, 1), lambda i: (0, i, 0)),
            pl.BlockSpec((1, D), lambda i: (0, 0)),
        ],
        out_specs=pl.BlockSpec((RB, D), lambda i: (i, 0)),
        out_shape=jax.ShapeDtypeStruct((N, D), jnp.float32),
    )(p, xwp, deg, b)


def kernel(x, edge_index, W1, b1, g1, be1, W2, b2, g2, be2, W3, b3):
    src = edge_index[0]
    dst = edge_index[1]
    pad = jnp.full((E_PAD - E,), N, dtype=jnp.int32)
    srcp = jnp.concatenate([src, pad]).reshape(NW, BPW, BLK)
    dstp = jnp.concatenate([dst, pad]).reshape(NW, BPW, BLK)
    x_pad = jnp.zeros((NPAD, D), jnp.float32).at[:N].set(x)
    ones_blk = jnp.ones((BLK,), jnp.float32)
    zeros_rows = jnp.zeros((RPT, D), jnp.float32)

    deg = _sc_degree(ones_blk, zeros_rows[:, 0], dstp).reshape(NC, NPAD, 1)

    b1r, g1r, be1r = b1.reshape(1, D), g1.reshape(1, D), be1.reshape(1, D)
    b2r, g2r, be2r = b2.reshape(1, D), g2.reshape(1, D), be2.reshape(1, D)
    b3r = b3.reshape(1, D)

    xw1 = _t1(x_pad, W1, deg)
    p1 = _sc_aggregate(xw1, zeros_rows, srcp, dstp)
    xw2 = _tmid(p1, xw1, deg, b1r, g1r, be1r, W2)
    p2 = _sc_aggregate(xw2, zeros_rows, srcp, dstp)
    xw3 = _tmid(p2, xw2, deg, b2r, g2r, be2r, W3)
    p3 = _sc_aggregate(xw3, zeros_rows, srcp, dstp)
    return _tlast(p3, xw3, deg, b3r)


# speedup vs baseline: 7.7085x; 1.1770x over previous
import functools

import jax
import jax.numpy as jnp
from jax import lax
from jax.experimental import pallas as pl
from jax.experimental.pallas import tpu as pltpu
from jax.experimental.pallas import tpu_sc as plsc

N = 10000
D = 128
E = 320000
EPS = 1e-5

NC = 2
NS = 16
NW = NC * NS

BLK = 128
BPW = 80
CHUNK = 16
NCHUNK = BPW // CHUNK
EPW = BPW * BLK
E_PAD = NW * EPW

NPAD = 10240
RPT = NPAD // NS
RB = 512
GB = NPAD // RB

_mesh = plsc.VectorSubcoreMesh(core_axis_name="c", subcore_axis_name="s")


@functools.partial(
    pl.kernel,
    out_type=jax.ShapeDtypeStruct((NC, NPAD), jnp.float32),
    mesh=_mesh,
    scratch_types=[
        pltpu.VMEM_SHARED((NPAD,), jnp.float32),
        pltpu.VMEM((BPW, BLK), jnp.int32),
        pltpu.VMEM((BLK,), jnp.float32),
    ],
)
def _sc_degree(ones_hbm, zeros_hbm, dstp_hbm, out_hbm, dacc, didx, ones_v):
    c = lax.axis_index("c")
    s = lax.axis_index("s")
    w = c * NS + s
    pltpu.sync_copy(dstp_hbm.at[w], didx)
    pltpu.sync_copy(ones_hbm, ones_v)
    pltpu.sync_copy(zeros_hbm, dacc.at[pl.ds(s * RPT, RPT)])
    plsc.subcore_barrier()

    def body(b, carry):
        pltpu.sync_copy(ones_v, dacc.at[didx.at[b]], add=True)
        return carry

    lax.fori_loop(0, BPW, body, 0)
    plsc.subcore_barrier()
    pltpu.sync_copy(dacc.at[pl.ds(s * RPT, RPT)], out_hbm.at[c, pl.ds(s * RPT, RPT)])


@functools.partial(
    pl.kernel,
    out_type=jax.ShapeDtypeStruct((NC, NPAD, D), jnp.float32),
    mesh=_mesh,
    scratch_types=[
        pltpu.VMEM_SHARED((NPAD, D), jnp.float32),
        pltpu.VMEM((CHUNK, BLK), jnp.int32),
        pltpu.VMEM((CHUNK, BLK), jnp.int32),
        pltpu.VMEM((BLK, D), jnp.float32),
        pltpu.VMEM((BLK, D), jnp.float32),
        pltpu.SemaphoreType.DMA,
        pltpu.SemaphoreType.DMA,
    ],
)
def _sc_aggregate(table_hbm, zeros_hbm, srcp_hbm, dstp_hbm, out_hbm,
                  accum, sidx, didx, buf0, buf1, sem0, sem1):
    c = lax.axis_index("c")
    s = lax.axis_index("s")
    w = c * NS + s
    pltpu.sync_copy(zeros_hbm, accum.at[pl.ds(s * RPT, RPT), :])
    plsc.subcore_barrier()

    def chunk_body(k, carry):
        pltpu.sync_copy(srcp_hbm.at[w, pl.ds(k * CHUNK, CHUNK)], sidx)
        pltpu.sync_copy(dstp_hbm.at[w, pl.ds(k * CHUNK, CHUNK)], didx)
        pltpu.async_copy(table_hbm.at[sidx.at[0]], buf0, sem0)

        def body(i, carry2):
            b1 = 2 * i + 1
            b2 = jnp.minimum(2 * i + 2, CHUNK - 1)
            pltpu.async_copy(table_hbm.at[sidx.at[b1]], buf1, sem1)
            pltpu.make_async_copy(table_hbm.at[sidx.at[0]], buf0, sem0).wait()
            pltpu.sync_copy(buf0, accum.at[didx.at[2 * i]], add=True)
            pltpu.async_copy(table_hbm.at[sidx.at[b2]], buf0, sem0)
            pltpu.make_async_copy(table_hbm.at[sidx.at[0]], buf1, sem1).wait()
            pltpu.sync_copy(buf1, accum.at[didx.at[b1]], add=True)
            return carry2

        lax.fori_loop(0, CHUNK // 2, body, 0)
        pltpu.make_async_copy(table_hbm.at[sidx.at[0]], buf0, sem0).wait()
        return carry

    lax.fori_loop(0, NCHUNK, chunk_body, 0)
    plsc.subcore_barrier()
    pltpu.sync_copy(accum.at[pl.ds(s * RPT, RPT), :],
                    out_hbm.at[c, pl.ds(s * RPT, RPT), :])


def _t1_body(x_ref, w_ref, deg_ref, o_ref):
    dinv = lax.rsqrt(deg_ref[0] + deg_ref[1] + 1.0)
    o_ref[...] = jnp.dot(x_ref[...], w_ref[...],
                         preferred_element_type=jnp.float32) * dinv


def _t1(x_pad, W, deg):
    return pl.pallas_call(
        _t1_body,
        grid=(GB,),
        in_specs=[
            pl.BlockSpec((RB, D), lambda i: (i, 0)),
            pl.BlockSpec((D, D), lambda i: (0, 0)),
            pl.BlockSpec((NC, RB, 1), lambda i: (0, i, 0)),
        ],
        out_specs=pl.BlockSpec((RB, D), lambda i: (i, 0)),
        out_shape=jax.ShapeDtypeStruct((NPAD, D), jnp.float32),
    )(x_pad, W, deg)


def _tmid_body(p_ref, xwp_ref, deg_ref, b_ref, g_ref, be_ref, wn_ref,
               o_ref, stats):
    t = pl.program_id(0)
    i = pl.program_id(1)
    dinv = lax.rsqrt(deg_ref[0] + deg_ref[1] + 1.0)
    y = (p_ref[0] + p_ref[1] + xwp_ref[...]) * dinv + b_ref[...]
    rows = jax.lax.broadcasted_iota(jnp.int32, (RB, 1), 0) + i * RB
    mask = rows < N

    @pl.when(t == 0)
    def _():
        @pl.when(i == 0)
        def _():
            stats[...] = jnp.zeros_like(stats)
        ym = jnp.where(mask, y, 0.0)
        stats[0:1, :] += jnp.sum(ym, axis=0, keepdims=True)
        stats[1:2, :] += jnp.sum(ym * ym, axis=0, keepdims=True)

    @pl.when(t == 1)
    def _():
        mean = stats[0:1, :] * (1.0 / N)
        var = stats[1:2, :] * (1.0 / N) - mean * mean
        rstd = lax.rsqrt(var + EPS)
        h = jnp.maximum((y - mean) * rstd * g_ref[...] + be_ref[...], 0.0)
        nxt = jnp.dot(h, wn_ref[...], preferred_element_type=jnp.float32)
        o_ref[...] = jnp.where(mask, nxt * dinv, 0.0)


def _tmid(p, xwp, deg, b, g, be, Wn):
    return pl.pallas_call(
        _tmid_body,
        grid=(2, GB),
        in_specs=[
            pl.BlockSpec((NC, RB, D), lambda t, i: (0, i, 0)),
            pl.BlockSpec((RB, D), lambda t, i: (i, 0)),
            pl.BlockSpec((NC, RB, 1), lambda t, i: (0, i, 0)),
            pl.BlockSpec((1, D), lambda t, i: (0, 0)),
            pl.BlockSpec((1, D), lambda t, i: (0, 0)),
            pl.BlockSpec((1, D), lambda t, i: (0, 0)),
            pl.BlockSpec((D, D), lambda t, i: (0, 0)),
        ],
        out_specs=pl.BlockSpec((RB, D), lambda t, i: (i, 0)),
        out_shape=jax.ShapeDtypeStruct((NPAD, D), jnp.float32),
        scratch_shapes=[pltpu.VMEM((8, D), jnp.float32)],
    )(p, xwp, deg, b, g, be, Wn)


def _tlast_body(p_ref, xwp_ref, deg_ref, b_ref, o_ref):
    dinv = lax.rsqrt(deg_ref[0] + deg_ref[1] + 1.0)
    o_ref[...] = (p_ref[0] + p_ref[1] + xwp_ref[...]) * dinv + b_ref[...]


def _tlast(p, xwp, deg, b):
    return pl.pallas_call(
        _tlast_body,
        grid=(GB,),
        in_specs=[
            pl.BlockSpec((NC, RB, D), lambda i: (0, i, 0)),
            pl.BlockSpec((RB, D), lambda i: (i, 0)),
            pl.BlockSpec((NC, RB, 1), lambda i: (0, i, 0)),
            pl.BlockSpec((1, D), lambda i: (0, 0)),
        ],
        out_specs=pl.BlockSpec((RB, D), lambda i: (i, 0)),
        out_shape=jax.ShapeDtypeStruct((N, D), jnp.float32),
    )(p, xwp, deg, b)


def kernel(x, edge_index, W1, b1, g1, be1, W2, b2, g2, be2, W3, b3):
    src = edge_index[0]
    dst = edge_index[1]
    pad = jnp.full((E_PAD - E,), N, dtype=jnp.int32)
    srcp = jnp.concatenate([src, pad]).reshape(NW, BPW, BLK)
    dstp = jnp.concatenate([dst, pad]).reshape(NW, BPW, BLK)
    pad_dst = N + (jnp.arange(E_PAD - E, dtype=jnp.int32) % (NPAD - N))
    dstp_agg = jnp.concatenate([dst, pad_dst]).reshape(NW, BPW, BLK)
    x_pad = jnp.zeros((NPAD, D), jnp.float32).at[:N].set(x)
    ones_blk = jnp.ones((BLK,), jnp.float32)
    zeros_rows = jnp.zeros((RPT, D), jnp.float32)

    deg = _sc_degree(ones_blk, zeros_rows[:, 0], dstp).reshape(NC, NPAD, 1)

    b1r, g1r, be1r = b1.reshape(1, D), g1.reshape(1, D), be1.reshape(1, D)
    b2r, g2r, be2r = b2.reshape(1, D), g2.reshape(1, D), be2.reshape(1, D)
    b3r = b3.reshape(1, D)

    xw1 = _t1(x_pad, W1, deg)
    p1 = _sc_aggregate(xw1, zeros_rows, srcp, dstp_agg)
    xw2 = _tmid(p1, xw1, deg, b1r, g1r, be1r, W2)
    p2 = _sc_aggregate(xw2, zeros_rows, srcp, dstp_agg)
    xw3 = _tmid(p2, xw2, deg, b2r, g2r, be2r, W3)
    p3 = _sc_aggregate(xw3, zeros_rows, srcp, dstp_agg)
    return _tlast(p3, xw3, deg, b3r)
